# baseline (speedup 1.0000x reference)
import functools

import jax
import jax.numpy as jnp
from jax import lax
from jax.experimental import pallas as pl
from jax.experimental.pallas import tpu as pltpu
from jax.experimental.pallas import tpu_sc as plsc

B = 50000
D = 256
V_PAD = 128
CHUNK = 128
NW = 32
LANES = 16
N_FULL = B // CHUNK
TAIL = B - N_FULL * CHUNK
BASE_CPW = N_FULL // NW
EXTRA = N_FULL - BASE_CPW * NW
MAX_CPW = BASE_CPW + 1


def _fill_rows(table_v, idx_v, buf, n_rows):
    def body(g, _):
        ivec = idx_v[pl.ds(g * LANES, LANES)] * D
        for lane in range(LANES):
            off = ivec[lane]
            dst = (g * LANES + lane) * D
            for k in range(D // LANES):
                buf[pl.ds(dst + k * LANES, LANES)] = (
                    table_v[pl.ds(off + k * LANES, LANES)])
        return 0
    lax.fori_loop(0, n_rows // LANES, body, 0)


def _lookup_kernel(idx_hbm, table_hbm, out_hbm,
                   table_v, idx_v, buf0, buf1, ss0, ss1):
    wid = lax.axis_index("s") * 2 + lax.axis_index("c")
    nc = BASE_CPW + jnp.where(wid < EXTRA, 1, 0)
    base_chunk = BASE_CPW * wid + jnp.minimum(wid, EXTRA)
    base_row = base_chunk * CHUNK

    bufs = (buf0, buf1)
    sem_s = (ss0, ss1)

    pltpu.sync_copy(table_hbm, table_v)

    def scatter(i):
        return pltpu.make_async_copy(
            bufs[i % 2],
            out_hbm.at[pl.ds((base_row + i * CHUNK) * D, CHUNK * D)],
            sem_s[i % 2])

    for i in range(MAX_CPW):
        @pl.when(i < nc)
        def _(i=i):
            pltpu.sync_copy(idx_hbm.at[pl.ds(base_row + i * CHUNK, CHUNK)],
                            idx_v)
            if i >= 2:
                scatter(i - 2).wait()
            _fill_rows(table_v, idx_v, bufs[i % 2], CHUNK)
            scatter(i).start()

    @pl.when(nc == BASE_CPW)
    def _():
        scatter(BASE_CPW - 2).wait()
        scatter(BASE_CPW - 1).wait()

    @pl.when(nc == MAX_CPW)
    def _():
        scatter(MAX_CPW - 2).wait()
        scatter(MAX_CPW - 1).wait()

    @pl.when(wid == NW - 1)
    def _():
        pltpu.sync_copy(idx_hbm.at[pl.ds(N_FULL * CHUNK, TAIL)],
                        idx_v.at[pl.ds(0, TAIL)])
        _fill_rows(table_v, idx_v, buf0, TAIL)
        pltpu.sync_copy(buf0.at[pl.ds(0, TAIL * D)],
                        out_hbm.at[pl.ds(N_FULL * CHUNK * D, TAIL * D)])


@jax.jit
def _run(atomic_numbers, table_flat):
    mesh = plsc.VectorSubcoreMesh(core_axis_name="c", subcore_axis_name="s")
    f = functools.partial(
        pl.kernel, mesh=mesh,
        out_type=jax.ShapeDtypeStruct((B * D,), jnp.float32),
        scratch_types=[
            pltpu.VMEM((V_PAD * D,), jnp.float32),
            pltpu.VMEM((CHUNK,), jnp.int32),
            pltpu.VMEM((CHUNK * D,), jnp.float32),
            pltpu.VMEM((CHUNK * D,), jnp.float32),
            pltpu.SemaphoreType.DMA,
            pltpu.SemaphoreType.DMA,
        ],
    )(_lookup_kernel)
    return f(atomic_numbers, table_flat)


def kernel(atomic_numbers, table):
    table_p = jnp.zeros((V_PAD, D), table.dtype).at[:table.shape[0]].set(table)
    out = _run(atomic_numbers.astype(jnp.int32), table_p.reshape(-1))
    return out.reshape(B, D)
